# baseline (speedup 1.0000x reference)
import functools

import jax
import jax.numpy as jnp
from jax.experimental import pallas as pl
from jax.experimental.pallas import tpu as pltpu


def _swiglu_ffn_kernel(x_ref, w1_ref, w2_ref, w3_ref, out_ref):
    k = pl.program_id(1)
    x = x_ref[0].astype(jnp.bfloat16)
    a = jnp.dot(x, w1_ref[0].astype(jnp.bfloat16),
                preferred_element_type=jnp.float32)
    b = jnp.dot(x, w3_ref[0].astype(jnp.bfloat16),
                preferred_element_type=jnp.float32)
    h = (a * jax.nn.sigmoid(a) * b).astype(jnp.bfloat16)
    p = jnp.dot(h, w2_ref[0].astype(jnp.bfloat16),
                preferred_element_type=jnp.float32)

    @pl.when(k == 0)
    def _init():
        out_ref[0] = p

    @pl.when(k != 0)
    def _acc():
        out_ref[0] += p


@functools.partial(jax.jit, static_argnames=("hblk",))
def _grouped_swiglu(x, w1, w2, w3, hblk=512):
    e, tok, dim = x.shape
    hid = w1.shape[2]
    kk = hid // hblk
    return pl.pallas_call(
        _swiglu_ffn_kernel,
        grid=(e, kk),
        in_specs=[
            pl.BlockSpec((1, tok, dim), lambda i, k: (i, 0, 0)),
            pl.BlockSpec((1, dim, hblk), lambda i, k: (i, 0, k)),
            pl.BlockSpec((1, hblk, dim), lambda i, k: (i, k, 0)),
            pl.BlockSpec((1, dim, hblk), lambda i, k: (i, 0, k)),
        ],
        out_specs=pl.BlockSpec((1, tok, dim), lambda i, k: (i, 0, 0)),
        out_shape=jax.ShapeDtypeStruct((e, tok, dim), jnp.float32),
        compiler_params=pltpu.CompilerParams(
            dimension_semantics=("arbitrary", "arbitrary"),
        ),
    )(x, w1, w2, w3)


def kernel(x, w1, w2, w3):
    return _grouped_swiglu(x, w1, w2, w3, hblk=256)


# speedup vs baseline: 1.0808x; 1.0808x over previous
import functools

import jax
import jax.numpy as jnp
from jax.experimental import pallas as pl
from jax.experimental.pallas import tpu as pltpu


def _swiglu_ffn_kernel(x_ref, w1_ref, w2_ref, w3_ref, out_ref):
    k = pl.program_id(1)
    x = x_ref[0].astype(jnp.bfloat16)
    a = jnp.dot(x, w1_ref[0].astype(jnp.bfloat16),
                preferred_element_type=jnp.float32)
    b = jnp.dot(x, w3_ref[0].astype(jnp.bfloat16),
                preferred_element_type=jnp.float32)
    h = (a * jax.nn.sigmoid(a) * b).astype(jnp.bfloat16)
    p = jnp.dot(h, w2_ref[0].astype(jnp.bfloat16),
                preferred_element_type=jnp.float32)

    @pl.when(k == 0)
    def _init():
        out_ref[0] = p

    @pl.when(k != 0)
    def _acc():
        out_ref[0] += p


@functools.partial(jax.jit, static_argnames=("hblk",))
def _grouped_swiglu(x, w1, w2, w3, hblk=512):
    e, tok, dim = x.shape
    hid = w1.shape[2]
    kk = hid // hblk
    return pl.pallas_call(
        _swiglu_ffn_kernel,
        grid=(e, kk),
        in_specs=[
            pl.BlockSpec((1, tok, dim), lambda i, k: (i, 0, 0)),
            pl.BlockSpec((1, dim, hblk), lambda i, k: (i, 0, k)),
            pl.BlockSpec((1, hblk, dim), lambda i, k: (i, k, 0)),
            pl.BlockSpec((1, dim, hblk), lambda i, k: (i, 0, k)),
        ],
        out_specs=pl.BlockSpec((1, tok, dim), lambda i, k: (i, 0, 0)),
        out_shape=jax.ShapeDtypeStruct((e, tok, dim), jnp.float32),
        compiler_params=pltpu.CompilerParams(
            dimension_semantics=("arbitrary", "arbitrary"),
        ),
    )(x, w1, w2, w3)


def kernel(x, w1, w2, w3):
    return _grouped_swiglu(x, w1, w2, w3, hblk=1024)


# speedup vs baseline: 1.0843x; 1.0032x over previous
import jax
import jax.numpy as jnp
from jax.experimental import pallas as pl
from jax.experimental.pallas import tpu as pltpu


def _swiglu_ffn_kernel(x_ref, w1_ref, w2_ref, w3_ref, out_ref):
    x = x_ref[0].astype(jnp.bfloat16)
    a = jnp.dot(x, w1_ref[0].astype(jnp.bfloat16),
                preferred_element_type=jnp.float32)
    b = jnp.dot(x, w3_ref[0].astype(jnp.bfloat16),
                preferred_element_type=jnp.float32)
    h = (a * jax.nn.sigmoid(a) * b).astype(jnp.bfloat16)
    out_ref[0] = jnp.dot(h, w2_ref[0].astype(jnp.bfloat16),
                         preferred_element_type=jnp.float32)


@jax.jit
def _grouped_swiglu(x, w1, w2, w3):
    e, tok, dim = x.shape
    hid = w1.shape[2]
    return pl.pallas_call(
        _swiglu_ffn_kernel,
        grid=(e,),
        in_specs=[
            pl.BlockSpec((1, tok, dim), lambda i: (i, 0, 0)),
            pl.BlockSpec((1, dim, hid), lambda i: (i, 0, 0)),
            pl.BlockSpec((1, hid, dim), lambda i: (i, 0, 0)),
            pl.BlockSpec((1, dim, hid), lambda i: (i, 0, 0)),
        ],
        out_specs=pl.BlockSpec((1, tok, dim), lambda i: (i, 0, 0)),
        out_shape=jax.ShapeDtypeStruct((e, tok, dim), jnp.float32),
        compiler_params=pltpu.CompilerParams(
            dimension_semantics=("parallel",),
        ),
    )(x, w1, w2, w3)


def kernel(x, w1, w2, w3):
    return _grouped_swiglu(x, w1, w2, w3)
